# baseline (speedup 1.0000x reference)
import functools

import jax
import jax.numpy as jnp
from jax.experimental import pallas as pl


def _interp_kernel(q8_ref, pt_ref, pn_ref, points_ref, out_ref, *, M):
    q = q8_ref[0, :, :]
    qn = jnp.sum(q * q, axis=1, keepdims=True)
    qp = jnp.dot(q, pt_ref[0, :, :], preferred_element_type=jnp.float32)
    d2 = (qp + pn_ref[0, :, :]) + qn

    v1 = jnp.min(d2, axis=1, keepdims=True)
    t = jnp.where(d2 > v1, d2, jnp.inf)
    v2 = jnp.min(t, axis=1, keepdims=True)
    t = jnp.where(t > v2, t, jnp.inf)
    v3 = jnp.min(t, axis=1, keepdims=True)

    inv = 1.0 / jnp.maximum(d2, 1e-10)
    masked = jnp.where(d2 <= v3, inv, 0.0)
    norm = jnp.sum(masked, axis=1, keepdims=True)

    acc = jnp.dot(masked, points_ref[0, :, :], preferred_element_type=jnp.float32)
    out_ref[0, :, :] = acc * (1.0 / norm)


@functools.partial(jax.jit, static_argnames=("tile_n",))
def _run(points, xyz1, xyz2, tile_n=1024):
    B, N, _ = xyz1.shape
    _, M, C = points.shape

    ctr = jnp.mean(xyz2, axis=1, keepdims=True)
    x1c = xyz1 - ctr
    x2c = xyz2 - ctr
    q8 = jnp.pad(x1c, ((0, 0), (0, 0), (0, 5)))
    pt = jnp.pad(-2.0 * jnp.transpose(x2c, (0, 2, 1)), ((0, 0), (0, 5), (0, 0)))
    pn = jnp.sum(x2c * x2c, axis=2)[:, None, :]

    grid = (B, N // tile_n)
    return pl.pallas_call(
        functools.partial(_interp_kernel, M=M),
        grid=grid,
        in_specs=[
            pl.BlockSpec((1, tile_n, 8), lambda b, n: (b, n, 0)),
            pl.BlockSpec((1, 8, M), lambda b, n: (b, 0, 0)),
            pl.BlockSpec((1, 1, M), lambda b, n: (b, 0, 0)),
            pl.BlockSpec((1, M, C), lambda b, n: (b, 0, 0)),
        ],
        out_specs=pl.BlockSpec((1, tile_n, C), lambda b, n: (b, n, 0)),
        out_shape=jax.ShapeDtypeStruct((B, N, C), jnp.float32),
    )(q8, pt, pn, points)


def kernel(points, xyz1, xyz2):
    return _run(points, xyz1, xyz2)


# speedup vs baseline: 1.0624x; 1.0624x over previous
import functools

import jax
import jax.numpy as jnp
from jax.experimental import pallas as pl
from jax.experimental.pallas import tpu as pltpu


def _interp_kernel(xyz1_ref, xyz2t_ref, points_ref, out_ref, *, M):
    qx = xyz1_ref[0, :, 0:1]
    qy = xyz1_ref[0, :, 1:2]
    qz = xyz1_ref[0, :, 2:3]
    px = xyz2t_ref[0, 0, :][None, :]
    py = xyz2t_ref[0, 1, :][None, :]
    pz = xyz2t_ref[0, 2, :][None, :]

    dx = qx - px
    dy = qy - py
    dz = qz - pz
    d2 = dx * dx + dy * dy + dz * dz

    v1 = jnp.min(d2, axis=1, keepdims=True)
    t = jnp.where(d2 > v1, d2, jnp.inf)
    v2 = jnp.min(t, axis=1, keepdims=True)
    t = jnp.where(t > v2, t, jnp.inf)
    v3 = jnp.min(t, axis=1, keepdims=True)

    inv = 1.0 / jnp.maximum(d2, 1e-10)
    masked = jnp.where(d2 <= v3, inv, 0.0)
    norm = jnp.sum(masked, axis=1, keepdims=True)

    acc = jnp.dot(masked, points_ref[0, :, :], preferred_element_type=jnp.float32)
    out_ref[0, :, :] = acc * (1.0 / norm)


@functools.partial(jax.jit, static_argnames=("tile_n",))
def _run(points, xyz1, xyz2, tile_n=2048):
    B, N, _ = xyz1.shape
    _, M, C = points.shape
    xyz2t = jnp.transpose(xyz2, (0, 2, 1))

    grid = (B, N // tile_n)
    return pl.pallas_call(
        functools.partial(_interp_kernel, M=M),
        grid=grid,
        in_specs=[
            pl.BlockSpec((1, tile_n, 3), lambda b, n: (b, n, 0)),
            pl.BlockSpec((1, 3, M), lambda b, n: (b, 0, 0)),
            pl.BlockSpec((1, M, C), lambda b, n: (b, 0, 0)),
        ],
        out_specs=pl.BlockSpec((1, tile_n, C), lambda b, n: (b, n, 0)),
        out_shape=jax.ShapeDtypeStruct((B, N, C), jnp.float32),
        compiler_params=pltpu.CompilerParams(
            dimension_semantics=("parallel", "arbitrary"),
        ),
    )(xyz1, xyz2t, points)


def kernel(points, xyz1, xyz2):
    return _run(points, xyz1, xyz2)
